# baseline (speedup 1.0000x reference)
import jax
import jax.numpy as jnp
from jax.experimental import pallas as pl


def _add_kernel(d_ref, p_ref, o_ref):
    o_ref[...] = d_ref[...] + p_ref[...]


def kernel(data, pos_emb_weight):
    B, S, E = data.shape
    W = S * E
    d2 = data.reshape(B, W)
    p2 = pos_emb_weight[:S].reshape(1, W)
    BLK = 32
    out = pl.pallas_call(
        _add_kernel,
        grid=(B // BLK,),
        in_specs=[
            pl.BlockSpec((BLK, W), lambda i: (i, 0)),
            pl.BlockSpec((1, W), lambda i: (0, 0)),
        ],
        out_specs=pl.BlockSpec((BLK, W), lambda i: (i, 0)),
        out_shape=jax.ShapeDtypeStruct((B, W), jnp.float32),
    )(d2, p2)
    return out.reshape(B, S, E)


# speedup vs baseline: 1.0474x; 1.0474x over previous
import functools
import jax
import jax.numpy as jnp
from jax import lax
from jax.experimental import pallas as pl
from jax.experimental.pallas import tpu as pltpu

R_CH = 64
NBUF = 4


def _body(n_ch, d_hbm, p_hbm, o_hbm, pos_v, bufs, in_sems, out_sems):
    pltpu.make_async_copy(p_hbm, pos_v, in_sems.at[NBUF]).start()
    pltpu.make_async_copy(p_hbm, pos_v, in_sems.at[NBUF]).wait()

    def rows_of(c):
        return pl.ds(c * R_CH, R_CH)

    def fire_in(b, c):
        pltpu.make_async_copy(
            d_hbm.at[rows_of(c)], bufs.at[b], in_sems.at[b]).start()

    def wait_in(b, c):
        pltpu.make_async_copy(
            d_hbm.at[rows_of(c)], bufs.at[b], in_sems.at[b]).wait()

    def fire_out(b, c):
        pltpu.make_async_copy(
            bufs.at[b], o_hbm.at[rows_of(c)], out_sems.at[b]).start()

    def wait_out(b, c):
        pltpu.make_async_copy(
            bufs.at[b], o_hbm.at[rows_of(c)], out_sems.at[b]).wait()

    def compute(b):
        bufs[b] = bufs[b] + pos_v[...]

    for b in range(NBUF - 1):
        fire_in(b, b)

    for b in range(NBUF):
        wait_in(b, b)
        compute(b)
        fire_out(b, b)
        bprev = (b + NBUF - 1) % NBUF
        if b == 0:
            fire_in(NBUF - 1, NBUF - 1)
        else:
            wait_out(bprev, b - 1)
            fire_in(bprev, b - 1 + NBUF)

    def grp(g, carry):
        for b in range(NBUF):
            c = g * NBUF + b
            wait_in(b, c)
            compute(b)
            fire_out(b, c)
            bprev = (b + NBUF - 1) % NBUF
            wait_out(bprev, c - 1)
            fire_in(bprev, jnp.minimum(c - 1 + NBUF, n_ch - 1))
        return carry

    lax.fori_loop(1, n_ch // NBUF, grp, 0, unroll=False)

    for b in range(NBUF - 1):
        wait_in(b, n_ch - 1)
    wait_out(NBUF - 1, n_ch - 1)


def kernel(data, pos_emb_weight):
    B, S, E = data.shape
    W = S * E
    d2 = data.reshape(B, W)
    p2 = pos_emb_weight[:S].reshape(1, W)
    n_ch = B // R_CH

    out = pl.pallas_call(
        functools.partial(_body, n_ch),
        in_specs=[
            pl.BlockSpec(memory_space=pltpu.HBM),
            pl.BlockSpec(memory_space=pltpu.HBM),
        ],
        out_specs=pl.BlockSpec(memory_space=pltpu.HBM),
        out_shape=jax.ShapeDtypeStruct((B, W), jnp.float32),
        scratch_shapes=[
            pltpu.VMEM((1, W), jnp.float32),
            pltpu.VMEM((NBUF, R_CH, W), jnp.float32),
            pltpu.SemaphoreType.DMA((NBUF + 1,)),
            pltpu.SemaphoreType.DMA((NBUF,)),
        ],
    )(d2, p2)
    return out.reshape(B, S, E)
